# baseline (speedup 1.0000x reference)
import functools

import jax
import jax.numpy as jnp
from jax import lax
from jax.experimental import pallas as pl
from jax.experimental.pallas import tpu as pltpu
from jax.experimental.pallas import tpu_sc as plsc


def _sc_posadd(x2, pos_table, B, S, H):
    NC, NS, L = 2, 16, 16
    NW = NC * NS
    seq_per_w = S // NW
    C = 8
    nsub = seq_per_w // C
    mesh = plsc.VectorSubcoreMesh(
        core_axis_name="c", subcore_axis_name="s", num_cores=NC)

    @functools.partial(
        pl.kernel,
        out_type=jax.ShapeDtypeStruct((B * S, H), jnp.float32),
        mesh=mesh,
        scratch_types=(
            [pltpu.VMEM((C, H), jnp.float32) for _ in range(2)]
            + [pltpu.VMEM((C, H), jnp.float32) for _ in range(2 * B)]
            + [pltpu.SemaphoreType.DMA for _ in range(2 + 4 * B)]
        ),
    )
    def k(x_hbm, pos_hbm, out_hbm, bp0, bp1, *rest):
        bufp = [bp0, bp1]
        bufx = [list(rest[:B]), list(rest[B:2 * B])]
        semp = list(rest[2 * B:2 * B + 2])
        semx = [list(rest[2 * B + 2:3 * B + 2]),
                list(rest[3 * B + 2:4 * B + 2])]
        semst = [list(rest[4 * B + 2:5 * B + 2]),
                 list(rest[5 * B + 2:6 * B + 2])]
        wid = lax.axis_index("s") * NC + lax.axis_index("c")
        s0 = wid * seq_per_w

        def start_pos(c):
            return pltpu.async_copy(
                pos_hbm.at[pl.ds(s0 + c * C, C)], bufp[c % 2], semp[c % 2])

        def start_load(c, b):
            return pltpu.async_copy(
                x_hbm.at[pl.ds(b * S + s0 + c * C, C)],
                bufx[c % 2][b], semx[c % 2][b])

        def start_store(c, b):
            return pltpu.async_copy(
                bufx[c % 2][b],
                out_hbm.at[pl.ds(b * S + s0 + c * C, C)], semst[c % 2][b])

        pos_copy = [None] * nsub
        in_copy = [[None] * B for _ in range(nsub)]
        st_copy = [[None] * B for _ in range(nsub)]
        pos_copy[0] = start_pos(0)
        if nsub > 1:
            pos_copy[1] = start_pos(1)
        for b in range(B):
            in_copy[0][b] = start_load(0, b)

        for c in range(nsub):
            j = c % 2
            pos_copy[c].wait()
            for b in range(B):
                in_copy[c][b].wait()

            bp, bxs = bufp[j], bufx[j]

            @plsc.parallel_loop(0, H, step=L)
            def _(o):
                for r in range(C):
                    pv = bp[r, pl.ds(o, L)]
                    for b in range(B):
                        bxs[b][r, pl.ds(o, L)] = bxs[b][r, pl.ds(o, L)] + pv

            for b in range(B):
                st_copy[c][b] = start_store(c, b)
            if c >= 1:
                for b in range(B):
                    st_copy[c - 1][b].wait()
            if c + 1 < nsub:
                for b in range(B):
                    in_copy[c + 1][b] = start_load(c + 1, b)
            if c + 2 < nsub:
                pos_copy[c + 2] = start_pos(c + 2)

        for b in range(B):
            st_copy[nsub - 1][b].wait()

    return k(x2, pos_table)


def kernel(x, pos_table):
    B, S, H = x.shape
    out2 = _sc_posadd(x.reshape(B * S, H), pos_table, B, S, H)
    return out2.reshape(B, S, H)
